# baseline (speedup 1.0000x reference)
import functools

import jax
import jax.numpy as jnp
from jax import lax
from jax.experimental import pallas as pl
from jax.experimental.pallas import tpu as pltpu
from jax.experimental.pallas import tpu_sc as plsc

B = 16384
N_MODELS = 1000
N_MODELS_PAD = 1024
N_QUERIES = 100000
DIM = 128
TDIM = 1536

NC = 2
NS = 16
LANES = 16
NW = NC * NS
RPW = B // NW
CHUNK = 16
NCHUNKS = RPW // CHUNK
NCOL = TDIM // LANES
VWORDS = TDIM // 2
NGRP = TDIM // 32

_BITREV = (0, 8, 4, 12, 2, 10, 6, 14, 1, 9, 5, 13, 3, 11, 7, 15)


def _v_table_body(p_ref, c_ref, w_ref, v_ref):
    p = p_ref[...]
    nrm = jnp.sqrt(jnp.sum(p * p, axis=1, keepdims=True))
    u = (p / jnp.maximum(nrm, 1e-12)) * c_ref[...]
    v = lax.dot_general(
        u, w_ref[...], (((1,), (0,)), ((), ())),
        preferred_element_type=jnp.float32,
        precision=lax.Precision.HIGHEST,
    )
    lo = lax.bitcast_convert_type(
        v[:, :VWORDS].astype(jnp.bfloat16), jnp.uint16).astype(jnp.int32)
    hi = lax.bitcast_convert_type(
        v[:, VWORDS:].astype(jnp.bfloat16), jnp.uint16).astype(jnp.int32)
    v_ref[...] = lo | (hi << 16)


_build_v_table = pl.pallas_call(
    _v_table_body,
    out_shape=jax.ShapeDtypeStruct((N_MODELS, VWORDS), jnp.int32),
)


def _sc_body(q_hbm, v_hbm, b_hbm, qid_hbm, mid_hbm, out_hbm,
             qidx, midx, qbuf0, qbuf1, vbuf0, vbuf1, obuf, bvmem,
             sq0, sq1, sv0, sv1):
    cid = lax.axis_index("c")
    sid = lax.axis_index("s")
    wid = sid * NC + cid
    base = wid * RPW

    pltpu.sync_copy(qid_hbm.at[pl.ds(base, RPW)], qidx)
    pltpu.sync_copy(mid_hbm.at[pl.ds(base, RPW)], midx)
    pltpu.sync_copy(b_hbm, bvmem)

    bvec = bvmem[...]
    iota = lax.broadcasted_iota(jnp.int32, (LANES,), 0)
    dnums = lax.GatherDimensionNumbers(
        offset_dims=(), collapsed_slice_dims=(0,), start_index_map=(0,))

    def rot(v, s):
        idx = ((iota + s) % LANES)[:, None]
        return lax.gather(v, idx, dnums, (1,),
                          mode=lax.GatherScatterMode.PROMISE_IN_BOUNDS)

    def combine(a, b, s):
        m = (iota & s) == 0
        return (jnp.where(m, a, rot(b, LANES - s))
                + jnp.where(m, rot(a, s), b))

    def fire(g, qb, vb, sq, sv):
        pltpu.async_copy(q_hbm.at[qidx.at[pl.ds(g * CHUNK, CHUNK)]], qb, sq)
        pltpu.async_copy(v_hbm.at[midx.at[pl.ds(g * CHUNK, CHUNK)]], vb, sv)

    def drain(qb, vb, sq, sv):
        pltpu.make_async_copy(q_hbm.at[pl.ds(0, CHUNK)], qb, sq).wait()
        pltpu.make_async_copy(v_hbm.at[pl.ds(0, CHUNK)], vb, sv).wait()

    def compute(g, qb, vb):
        def col(jj, accs):
            new = list(accs)
            for u in range(2):
                j = jj * 2 + u
                for r in range(CHUNK):
                    w = vb[r, pl.ds(j * LANES, LANES)]
                    lo = lax.bitcast_convert_type(w << 16, jnp.float32)
                    hi = lax.bitcast_convert_type(w & jnp.int32(-65536),
                                                  jnp.float32)
                    qa = qb[r, pl.ds(j * LANES, LANES)]
                    qc = qb[r, pl.ds(VWORDS + j * LANES, LANES)]
                    new[r] = new[r] + qa * lo + qc * hi
            return tuple(new)

        accs = lax.fori_loop(
            0, NGRP // 2, col,
            tuple(jnp.zeros((LANES,), jnp.float32) for _ in range(CHUNK)))

        vs = [accs[_BITREV[k]] for k in range(CHUNK)]
        for s in (8, 4, 2, 1):
            vs = [combine(vs[2 * i], vs[2 * i + 1], s)
                  for i in range(len(vs) // 2)]
        obuf[pl.ds(g * CHUNK, CHUNK)] = vs[0] + bvec

    fire(0, qbuf0, vbuf0, sq0, sv0)
    fire(1, qbuf1, vbuf1, sq1, sv1)

    def pipe(gg, _):
        g0 = 2 * gg
        drain(qbuf0, vbuf0, sq0, sv0)
        compute(g0, qbuf0, vbuf0)
        fire(g0 + 2, qbuf0, vbuf0, sq0, sv0)
        g1 = 2 * gg + 1
        drain(qbuf1, vbuf1, sq1, sv1)
        compute(g1, qbuf1, vbuf1)
        fire(g1 + 2, qbuf1, vbuf1, sq1, sv1)
        return 0

    lax.fori_loop(0, NCHUNKS // 2 - 1, pipe, 0)
    drain(qbuf0, vbuf0, sq0, sv0)
    compute(NCHUNKS - 2, qbuf0, vbuf0)
    drain(qbuf1, vbuf1, sq1, sv1)
    compute(NCHUNKS - 1, qbuf1, vbuf1)

    pltpu.sync_copy(obuf, out_hbm.at[pl.ds(base, RPW)])


_sc_dot = pl.kernel(
    _sc_body,
    out_type=jax.ShapeDtypeStruct((B,), jnp.float32),
    mesh=plsc.VectorSubcoreMesh(core_axis_name="c", subcore_axis_name="s"),
    scratch_types=[
        pltpu.VMEM((RPW,), jnp.int32),
        pltpu.VMEM((RPW,), jnp.int32),
        pltpu.VMEM((CHUNK, TDIM), jnp.float32),
        pltpu.VMEM((CHUNK, TDIM), jnp.float32),
        pltpu.VMEM((CHUNK, VWORDS), jnp.int32),
        pltpu.VMEM((CHUNK, VWORDS), jnp.int32),
        pltpu.VMEM((RPW,), jnp.float32),
        pltpu.VMEM((LANES,), jnp.float32),
        pltpu.SemaphoreType.DMA,
        pltpu.SemaphoreType.DMA,
        pltpu.SemaphoreType.DMA,
        pltpu.SemaphoreType.DMA,
    ],
)


def kernel(P_weight, Q_weight, text_proj_W, classifier_W, classifier_b,
           query_ids, model_ids):
    v_packed = _build_v_table(P_weight, classifier_W, text_proj_W)
    b16 = jnp.broadcast_to(classifier_b.astype(jnp.float32), (LANES,))
    return _sc_dot(Q_weight, v_packed, b16, query_ids, model_ids)


# speedup vs baseline: 1.3117x; 1.3117x over previous
import functools

import jax
import jax.numpy as jnp
from jax import lax
from jax.experimental import pallas as pl
from jax.experimental.pallas import tpu as pltpu
from jax.experimental.pallas import tpu_sc as plsc

B = 16384
N_MODELS = 1000
N_MODELS_PAD = 1024
N_QUERIES = 100000
DIM = 128
TDIM = 1536

NC = 2
NS = 16
LANES = 16
NW = NC * NS
RPW = B // NW
CHUNK = 16
NCHUNKS = RPW // CHUNK
NCOL = TDIM // LANES
VWORDS = TDIM // 2
NGRP = TDIM // 32

_BITREV = (0, 8, 4, 12, 2, 10, 6, 14, 1, 9, 5, 13, 3, 11, 7, 15)


def _v_table_body(p_ref, c_ref, w_ref, v_ref):
    p = p_ref[...]
    nrm = jnp.sqrt(jnp.sum(p * p, axis=1, keepdims=True))
    u = (p / jnp.maximum(nrm, 1e-12)) * c_ref[...]
    v = lax.dot_general(
        u, w_ref[...], (((1,), (0,)), ((), ())),
        preferred_element_type=jnp.float32,
        precision=lax.Precision.HIGHEST,
    )
    lo = lax.bitcast_convert_type(
        v[:, :VWORDS].astype(jnp.bfloat16), jnp.uint16).astype(jnp.int32)
    hi = lax.bitcast_convert_type(
        v[:, VWORDS:].astype(jnp.bfloat16), jnp.uint16).astype(jnp.int32)
    v_ref[...] = lo | (hi << 16)


_build_v_table = pl.pallas_call(
    _v_table_body,
    out_shape=jax.ShapeDtypeStruct((N_MODELS, VWORDS), jnp.int32),
)


def _sc_body(q_hbm, v_hbm, b_hbm, qid_hbm, mid_hbm, out_hbm,
             qidx, midx, qbuf0, qbuf1, vbuf0, vbuf1, obuf, bvmem,
             sq0, sq1, sv0, sv1):
    cid = lax.axis_index("c")
    sid = lax.axis_index("s")
    wid = sid * NC + cid
    base = wid * RPW

    pltpu.sync_copy(qid_hbm.at[pl.ds(base, RPW)], qidx)
    pltpu.sync_copy(mid_hbm.at[pl.ds(base, RPW)], midx)
    pltpu.sync_copy(b_hbm, bvmem)

    bvec = bvmem[...]
    iota = lax.broadcasted_iota(jnp.int32, (LANES,), 0)
    dnums = lax.GatherDimensionNumbers(
        offset_dims=(), collapsed_slice_dims=(0,), start_index_map=(0,))

    def rot(v, s):
        idx = ((iota + s) % LANES)[:, None]
        return lax.gather(v, idx, dnums, (1,),
                          mode=lax.GatherScatterMode.PROMISE_IN_BOUNDS)

    def combine(a, b, s):
        m = (iota & s) == 0
        return (jnp.where(m, a, rot(b, LANES - s))
                + jnp.where(m, rot(a, s), b))

    def fire(g, qb, vb, sq, sv):
        pltpu.async_copy(q_hbm.at[qidx.at[pl.ds(g * CHUNK, CHUNK)]], qb, sq)
        pltpu.async_copy(v_hbm.at[midx.at[pl.ds(g * CHUNK, CHUNK)]], vb, sv)

    def drain(qb, vb, sq, sv):
        pltpu.make_async_copy(q_hbm.at[pl.ds(0, CHUNK)], qb, sq).wait()
        pltpu.make_async_copy(v_hbm.at[pl.ds(0, CHUNK)], vb, sv).wait()

    def compute(g, qb, vb):
        def col(j, accs):
            new = []
            for r in range(CHUNK):
                w = vb[r, pl.ds(j * LANES, LANES)]
                lo = lax.bitcast_convert_type(w << 16, jnp.float32)
                hi = lax.bitcast_convert_type(w & jnp.int32(-65536),
                                              jnp.float32)
                qa = qb[r, pl.ds(j * LANES, LANES)]
                qc = qb[r, pl.ds(VWORDS + j * LANES, LANES)]
                new.append(accs[r] + qa * lo + qc * hi)
            return tuple(new)

        accs = lax.fori_loop(
            0, NGRP, col,
            tuple(jnp.zeros((LANES,), jnp.float32) for _ in range(CHUNK)))

        vs = [accs[_BITREV[k]] for k in range(CHUNK)]
        for s in (8, 4, 2, 1):
            vs = [combine(vs[2 * i], vs[2 * i + 1], s)
                  for i in range(len(vs) // 2)]
        obuf[pl.ds(g * CHUNK, CHUNK)] = vs[0] + bvec

    fire(0, qbuf0, vbuf0, sq0, sv0)
    fire(1, qbuf1, vbuf1, sq1, sv1)

    def pipe(gg, _):
        g0 = 2 * gg
        drain(qbuf0, vbuf0, sq0, sv0)
        compute(g0, qbuf0, vbuf0)
        fire(g0 + 2, qbuf0, vbuf0, sq0, sv0)
        g1 = 2 * gg + 1
        drain(qbuf1, vbuf1, sq1, sv1)
        compute(g1, qbuf1, vbuf1)
        fire(g1 + 2, qbuf1, vbuf1, sq1, sv1)
        return 0

    lax.fori_loop(0, NCHUNKS // 2 - 1, pipe, 0)
    drain(qbuf0, vbuf0, sq0, sv0)
    compute(NCHUNKS - 2, qbuf0, vbuf0)
    drain(qbuf1, vbuf1, sq1, sv1)
    compute(NCHUNKS - 1, qbuf1, vbuf1)

    pltpu.sync_copy(obuf, out_hbm.at[pl.ds(base, RPW)])


_sc_dot = pl.kernel(
    _sc_body,
    out_type=jax.ShapeDtypeStruct((B,), jnp.float32),
    mesh=plsc.VectorSubcoreMesh(core_axis_name="c", subcore_axis_name="s"),
    scratch_types=[
        pltpu.VMEM((RPW,), jnp.int32),
        pltpu.VMEM((RPW,), jnp.int32),
        pltpu.VMEM((CHUNK, TDIM), jnp.float32),
        pltpu.VMEM((CHUNK, TDIM), jnp.float32),
        pltpu.VMEM((CHUNK, VWORDS), jnp.int32),
        pltpu.VMEM((CHUNK, VWORDS), jnp.int32),
        pltpu.VMEM((RPW,), jnp.float32),
        pltpu.VMEM((LANES,), jnp.float32),
        pltpu.SemaphoreType.DMA,
        pltpu.SemaphoreType.DMA,
        pltpu.SemaphoreType.DMA,
        pltpu.SemaphoreType.DMA,
    ],
)


def kernel(P_weight, Q_weight, text_proj_W, classifier_W, classifier_b,
           query_ids, model_ids):
    v_packed = _build_v_table(P_weight, classifier_W, text_proj_W)
    b16 = jnp.broadcast_to(classifier_b.astype(jnp.float32), (LANES,))
    return _sc_dot(Q_weight, v_packed, b16, query_ids, model_ids)


# speedup vs baseline: 1.4981x; 1.1421x over previous
import functools

import jax
import jax.numpy as jnp
from jax import lax
from jax.experimental import pallas as pl
from jax.experimental.pallas import tpu as pltpu
from jax.experimental.pallas import tpu_sc as plsc

B = 16384
N_MODELS = 1000
N_MODELS_PAD = 1024
N_QUERIES = 100000
DIM = 128
TDIM = 1536

NC = 2
NS = 16
LANES = 16
NW = NC * NS
RPW = B // NW
CHUNK = 16
NCHUNKS = RPW // CHUNK
NCOL = TDIM // LANES
VWORDS = TDIM // 2
NGRP = TDIM // 32

_BITREV = (0, 8, 4, 12, 2, 10, 6, 14, 1, 9, 5, 13, 3, 11, 7, 15)


def _v_table_body(p_ref, c_ref, w_ref, v_ref):
    p = p_ref[...]
    nrm = jnp.sqrt(jnp.sum(p * p, axis=1, keepdims=True))
    u = (p / jnp.maximum(nrm, 1e-12)) * c_ref[...]
    v = lax.dot_general(
        u, w_ref[...], (((1,), (0,)), ((), ())),
        preferred_element_type=jnp.float32,
        precision=lax.Precision.HIGHEST,
    )
    lo = lax.bitcast_convert_type(
        v[:, :VWORDS].astype(jnp.bfloat16), jnp.uint16).astype(jnp.int32)
    hi = lax.bitcast_convert_type(
        v[:, VWORDS:].astype(jnp.bfloat16), jnp.uint16).astype(jnp.int32)
    v_ref[...] = lo | (hi << 16)


_build_v_table = pl.pallas_call(
    _v_table_body,
    out_shape=jax.ShapeDtypeStruct((N_MODELS, VWORDS), jnp.int32),
)


def _sc_body(q_hbm, v_hbm, b_hbm, qid_hbm, mid_hbm, out_hbm,
             qidx, midx, qbuf0, qbuf1, qbuf2, vbuf0, vbuf1, vbuf2,
             obuf, bvmem, sq0, sq1, sq2, sv0, sv1, sv2):
    cid = lax.axis_index("c")
    sid = lax.axis_index("s")
    wid = sid * NC + cid
    base = wid * RPW

    pltpu.sync_copy(qid_hbm.at[pl.ds(base, RPW)], qidx)
    pltpu.sync_copy(mid_hbm.at[pl.ds(base, RPW)], midx)
    pltpu.sync_copy(b_hbm, bvmem)

    bvec = bvmem[...]
    iota = lax.broadcasted_iota(jnp.int32, (LANES,), 0)
    dnums = lax.GatherDimensionNumbers(
        offset_dims=(), collapsed_slice_dims=(0,), start_index_map=(0,))

    def rot(v, s):
        idx = ((iota + s) % LANES)[:, None]
        return lax.gather(v, idx, dnums, (1,),
                          mode=lax.GatherScatterMode.PROMISE_IN_BOUNDS)

    def combine(a, b, s):
        m = (iota & s) == 0
        return (jnp.where(m, a, rot(b, LANES - s))
                + jnp.where(m, rot(a, s), b))

    def fire(g, qb, vb, sq, sv):
        pltpu.async_copy(q_hbm.at[qidx.at[pl.ds(g * CHUNK, CHUNK)]], qb, sq)
        pltpu.async_copy(v_hbm.at[midx.at[pl.ds(g * CHUNK, CHUNK)]], vb, sv)

    def drain(qb, vb, sq, sv):
        pltpu.make_async_copy(q_hbm.at[pl.ds(0, CHUNK)], qb, sq).wait()
        pltpu.make_async_copy(v_hbm.at[pl.ds(0, CHUNK)], vb, sv).wait()

    def compute(g, qb, vb):
        def col(j, accs):
            new = []
            for r in range(CHUNK):
                w = vb[r, pl.ds(j * LANES, LANES)]
                lo = lax.bitcast_convert_type(w << 16, jnp.float32)
                hi = lax.bitcast_convert_type(w & jnp.int32(-65536),
                                              jnp.float32)
                qa = qb[r, pl.ds(j * LANES, LANES)]
                qc = qb[r, pl.ds(VWORDS + j * LANES, LANES)]
                new.append(accs[r] + qa * lo + qc * hi)
            return tuple(new)

        accs = lax.fori_loop(
            0, NGRP, col,
            tuple(jnp.zeros((LANES,), jnp.float32) for _ in range(CHUNK)))

        vs = [accs[_BITREV[k]] for k in range(CHUNK)]
        for s in (8, 4, 2, 1):
            vs = [combine(vs[2 * i], vs[2 * i + 1], s)
                  for i in range(len(vs) // 2)]
        obuf[pl.ds(g * CHUNK, CHUNK)] = vs[0] + bvec

    bufs = ((qbuf0, vbuf0, sq0, sv0),
            (qbuf1, vbuf1, sq1, sv1),
            (qbuf2, vbuf2, sq2, sv2))
    for b in range(3):
        fire(b, *bufs[b])

    def pipe(gg, _):
        for b in range(3):
            g = 3 * gg + b
            drain(*bufs[b])
            compute(g, bufs[b][0], bufs[b][1])
            fire(g + 3, *bufs[b])
        return 0

    n_loop = (NCHUNKS - 3) // 3
    lax.fori_loop(0, n_loop, pipe, 0)
    for g in range(3 * n_loop, NCHUNKS):
        b = g % 3
        drain(*bufs[b])
        compute(g, bufs[b][0], bufs[b][1])
        if g + 3 < NCHUNKS:
            fire(g + 3, *bufs[b])

    pltpu.sync_copy(obuf, out_hbm.at[pl.ds(base, RPW)])


_sc_dot = pl.kernel(
    _sc_body,
    out_type=jax.ShapeDtypeStruct((B,), jnp.float32),
    mesh=plsc.VectorSubcoreMesh(core_axis_name="c", subcore_axis_name="s"),
    scratch_types=[
        pltpu.VMEM((RPW,), jnp.int32),
        pltpu.VMEM((RPW,), jnp.int32),
        pltpu.VMEM((CHUNK, TDIM), jnp.float32),
        pltpu.VMEM((CHUNK, TDIM), jnp.float32),
        pltpu.VMEM((CHUNK, TDIM), jnp.float32),
        pltpu.VMEM((CHUNK, VWORDS), jnp.int32),
        pltpu.VMEM((CHUNK, VWORDS), jnp.int32),
        pltpu.VMEM((CHUNK, VWORDS), jnp.int32),
        pltpu.VMEM((RPW,), jnp.float32),
        pltpu.VMEM((LANES,), jnp.float32),
        pltpu.SemaphoreType.DMA,
        pltpu.SemaphoreType.DMA,
        pltpu.SemaphoreType.DMA,
        pltpu.SemaphoreType.DMA,
        pltpu.SemaphoreType.DMA,
        pltpu.SemaphoreType.DMA,
    ],
)


def kernel(P_weight, Q_weight, text_proj_W, classifier_W, classifier_b,
           query_ids, model_ids):
    v_packed = _build_v_table(P_weight, classifier_W, text_proj_W)
    b16 = jnp.broadcast_to(classifier_b.astype(jnp.float32), (LANES,))
    return _sc_dot(Q_weight, v_packed, b16, query_ids, model_ids)


# speedup vs baseline: 1.5757x; 1.0518x over previous
import functools

import jax
import jax.numpy as jnp
from jax import lax
from jax.experimental import pallas as pl
from jax.experimental.pallas import tpu as pltpu
from jax.experimental.pallas import tpu_sc as plsc

B = 16384
N_MODELS = 1000
N_MODELS_PAD = 1024
N_QUERIES = 100000
DIM = 128
TDIM = 1536

NC = 2
NS = 16
LANES = 16
NW = NC * NS
RPW = B // NW
CHUNK = 16
NCHUNKS = RPW // CHUNK
NCOL = TDIM // LANES
VWORDS = TDIM // 2
NGRP = TDIM // 32

_BITREV = (0, 8, 4, 12, 2, 10, 6, 14, 1, 9, 5, 13, 3, 11, 7, 15)


def _v_table_body(p_ref, c_ref, w_ref, v_ref):
    p = p_ref[...]
    nrm = jnp.sqrt(jnp.sum(p * p, axis=1, keepdims=True))
    u = (p / jnp.maximum(nrm, 1e-12)) * c_ref[...]
    v = lax.dot_general(
        u, w_ref[...], (((1,), (0,)), ((), ())),
        preferred_element_type=jnp.float32,
        precision=lax.Precision.DEFAULT,
    )
    lo = lax.bitcast_convert_type(
        v[:, :VWORDS].astype(jnp.bfloat16), jnp.uint16).astype(jnp.int32)
    hi = lax.bitcast_convert_type(
        v[:, VWORDS:].astype(jnp.bfloat16), jnp.uint16).astype(jnp.int32)
    v_ref[...] = lo | (hi << 16)


_build_v_table = pl.pallas_call(
    _v_table_body,
    out_shape=jax.ShapeDtypeStruct((N_MODELS, VWORDS), jnp.int32),
)


def _sc_body(q_hbm, v_hbm, b_hbm, qid_hbm, mid_hbm, out_hbm,
             qidx, midx, qbuf0, qbuf1, qbuf2, vbuf0, vbuf1, vbuf2,
             obuf, bvmem, sq0, sq1, sq2, sv0, sv1, sv2):
    cid = lax.axis_index("c")
    sid = lax.axis_index("s")
    wid = sid * NC + cid
    base = wid * RPW

    pltpu.sync_copy(qid_hbm.at[pl.ds(base, RPW)], qidx)
    pltpu.sync_copy(mid_hbm.at[pl.ds(base, RPW)], midx)
    pltpu.sync_copy(b_hbm, bvmem)

    bvec = bvmem[...]
    iota = lax.broadcasted_iota(jnp.int32, (LANES,), 0)
    dnums = lax.GatherDimensionNumbers(
        offset_dims=(), collapsed_slice_dims=(0,), start_index_map=(0,))

    def rot(v, s):
        idx = ((iota + s) % LANES)[:, None]
        return lax.gather(v, idx, dnums, (1,),
                          mode=lax.GatherScatterMode.PROMISE_IN_BOUNDS)

    def combine(a, b, s):
        m = (iota & s) == 0
        return (jnp.where(m, a, rot(b, LANES - s))
                + jnp.where(m, rot(a, s), b))

    def fire(g, qb, vb, sq, sv):
        pltpu.async_copy(q_hbm.at[qidx.at[pl.ds(g * CHUNK, CHUNK)]], qb, sq)
        pltpu.async_copy(v_hbm.at[midx.at[pl.ds(g * CHUNK, CHUNK)]], vb, sv)

    def drain(qb, vb, sq, sv):
        pltpu.make_async_copy(q_hbm.at[pl.ds(0, CHUNK)], qb, sq).wait()
        pltpu.make_async_copy(v_hbm.at[pl.ds(0, CHUNK)], vb, sv).wait()

    def compute(g, qb, vb):
        def col(j, accs):
            new = []
            for r in range(CHUNK):
                w = vb[r, pl.ds(j * LANES, LANES)]
                lo = lax.bitcast_convert_type(w << 16, jnp.float32)
                hi = lax.bitcast_convert_type(w & jnp.int32(-65536),
                                              jnp.float32)
                qa = qb[r, pl.ds(j * LANES, LANES)]
                qc = qb[r, pl.ds(VWORDS + j * LANES, LANES)]
                new.append(accs[r] + qa * lo + qc * hi)
            return tuple(new)

        accs = lax.fori_loop(
            0, NGRP, col,
            tuple(jnp.zeros((LANES,), jnp.float32) for _ in range(CHUNK)))

        vs = [accs[_BITREV[k]] for k in range(CHUNK)]
        for s in (8, 4, 2, 1):
            vs = [combine(vs[2 * i], vs[2 * i + 1], s)
                  for i in range(len(vs) // 2)]
        obuf[pl.ds(g * CHUNK, CHUNK)] = vs[0] + bvec

    bufs = ((qbuf0, vbuf0, sq0, sv0),
            (qbuf1, vbuf1, sq1, sv1),
            (qbuf2, vbuf2, sq2, sv2))
    for b in range(3):
        fire(b, *bufs[b])

    def pipe(gg, _):
        for b in range(3):
            g = 3 * gg + b
            drain(*bufs[b])
            compute(g, bufs[b][0], bufs[b][1])
            fire(g + 3, *bufs[b])
        return 0

    n_loop = (NCHUNKS - 3) // 3
    lax.fori_loop(0, n_loop, pipe, 0)
    for g in range(3 * n_loop, NCHUNKS):
        b = g % 3
        drain(*bufs[b])
        compute(g, bufs[b][0], bufs[b][1])
        if g + 3 < NCHUNKS:
            fire(g + 3, *bufs[b])

    pltpu.sync_copy(obuf, out_hbm.at[pl.ds(base, RPW)])


_sc_dot = pl.kernel(
    _sc_body,
    out_type=jax.ShapeDtypeStruct((B,), jnp.float32),
    mesh=plsc.VectorSubcoreMesh(core_axis_name="c", subcore_axis_name="s"),
    scratch_types=[
        pltpu.VMEM((RPW,), jnp.int32),
        pltpu.VMEM((RPW,), jnp.int32),
        pltpu.VMEM((CHUNK, TDIM), jnp.float32),
        pltpu.VMEM((CHUNK, TDIM), jnp.float32),
        pltpu.VMEM((CHUNK, TDIM), jnp.float32),
        pltpu.VMEM((CHUNK, VWORDS), jnp.int32),
        pltpu.VMEM((CHUNK, VWORDS), jnp.int32),
        pltpu.VMEM((CHUNK, VWORDS), jnp.int32),
        pltpu.VMEM((RPW,), jnp.float32),
        pltpu.VMEM((LANES,), jnp.float32),
        pltpu.SemaphoreType.DMA,
        pltpu.SemaphoreType.DMA,
        pltpu.SemaphoreType.DMA,
        pltpu.SemaphoreType.DMA,
        pltpu.SemaphoreType.DMA,
        pltpu.SemaphoreType.DMA,
    ],
)


def kernel(P_weight, Q_weight, text_proj_W, classifier_W, classifier_b,
           query_ids, model_ids):
    v_packed = _build_v_table(P_weight, classifier_W, text_proj_W)
    b16 = jnp.broadcast_to(classifier_b.astype(jnp.float32), (LANES,))
    return _sc_dot(Q_weight, v_packed, b16, query_ids, model_ids)


# speedup vs baseline: 1.5787x; 1.0019x over previous
import functools

import jax
import jax.numpy as jnp
from jax import lax
from jax.experimental import pallas as pl
from jax.experimental.pallas import tpu as pltpu
from jax.experimental.pallas import tpu_sc as plsc

B = 16384
N_MODELS = 1000
N_MODELS_PAD = 1024
N_QUERIES = 100000
DIM = 128
TDIM = 1536

NC = 2
NS = 16
LANES = 16
NW = NC * NS
RPW = B // NW
CHUNK = 16
NCHUNKS = RPW // CHUNK
NCOL = TDIM // LANES
VWORDS = TDIM // 2
NGRP = TDIM // 32

_BITREV = (0, 8, 4, 12, 2, 10, 6, 14, 1, 9, 5, 13, 3, 11, 7, 15)


def _v_table_body(p_ref, c_ref, w_ref, v_ref):
    p = p_ref[...]
    nrm = jnp.sqrt(jnp.sum(p * p, axis=1, keepdims=True))
    u = (p / jnp.maximum(nrm, 1e-12)) * c_ref[...]
    v = lax.dot_general(
        u, w_ref[...], (((1,), (0,)), ((), ())),
        preferred_element_type=jnp.float32,
        precision=lax.Precision.DEFAULT,
    )
    lo = lax.bitcast_convert_type(
        v[:, :VWORDS].astype(jnp.bfloat16), jnp.uint16).astype(jnp.int32)
    hi = lax.bitcast_convert_type(
        v[:, VWORDS:].astype(jnp.bfloat16), jnp.uint16).astype(jnp.int32)
    v_ref[...] = lo | (hi << 16)


_build_v_table = pl.pallas_call(
    _v_table_body,
    out_shape=jax.ShapeDtypeStruct((N_MODELS, VWORDS), jnp.int32),
)


def _sc_body(q_hbm, v_hbm, b_hbm, qid_hbm, mid_hbm, out_hbm,
             qidx, midx, qbuf0, qbuf1, qbuf2, vbuf0, vbuf1, vbuf2,
             obuf, bvmem, sq0, sq1, sq2, sv0, sv1, sv2):
    cid = lax.axis_index("c")
    sid = lax.axis_index("s")
    wid = sid * NC + cid
    base = wid * RPW

    pltpu.sync_copy(qid_hbm.at[pl.ds(base, RPW)], qidx)
    pltpu.sync_copy(mid_hbm.at[pl.ds(base, RPW)], midx)
    pltpu.sync_copy(b_hbm, bvmem)

    bvec = bvmem[...]
    iota = lax.broadcasted_iota(jnp.int32, (LANES,), 0)
    dnums = lax.GatherDimensionNumbers(
        offset_dims=(), collapsed_slice_dims=(0,), start_index_map=(0,))

    def rot(v, s):
        idx = ((iota + s) % LANES)[:, None]
        return lax.gather(v, idx, dnums, (1,),
                          mode=lax.GatherScatterMode.PROMISE_IN_BOUNDS)

    def combine(a, b, s):
        m = (iota & s) == 0
        return (jnp.where(m, a, rot(b, LANES - s))
                + jnp.where(m, rot(a, s), b))

    def fire(g, qb, vb, sq, sv):
        pltpu.async_copy(q_hbm.at[qidx.at[pl.ds(g * CHUNK, CHUNK)]], qb, sq)
        pltpu.async_copy(v_hbm.at[midx.at[pl.ds(g * CHUNK, CHUNK)]], vb, sv)

    def drain(qb, vb, sq, sv):
        pltpu.make_async_copy(q_hbm.at[pl.ds(0, CHUNK)], qb, sq).wait()
        pltpu.make_async_copy(v_hbm.at[pl.ds(0, CHUNK)], vb, sv).wait()

    def compute(g, qb, vb):
        def col(j, accs):
            new = []
            for r in range(CHUNK):
                w = vb[r, pl.ds(j * LANES, LANES)]
                lo = lax.bitcast_convert_type(w << 16, jnp.float32)
                hi = lax.bitcast_convert_type(w & jnp.int32(-65536),
                                              jnp.float32)
                qa = qb[r, pl.ds(j * LANES, LANES)]
                qc = qb[r, pl.ds(VWORDS + j * LANES, LANES)]
                new.append(accs[r] + qa * lo + qc * hi)
            return tuple(new)

        accs = plsc.parallel_loop(
            0, NGRP,
            carry=tuple(jnp.zeros((LANES,), jnp.float32)
                        for _ in range(CHUNK)))(col)

        vs = [accs[_BITREV[k]] for k in range(CHUNK)]
        for s in (8, 4, 2, 1):
            vs = [combine(vs[2 * i], vs[2 * i + 1], s)
                  for i in range(len(vs) // 2)]
        obuf[pl.ds(g * CHUNK, CHUNK)] = vs[0] + bvec

    bufs = ((qbuf0, vbuf0, sq0, sv0),
            (qbuf1, vbuf1, sq1, sv1),
            (qbuf2, vbuf2, sq2, sv2))
    for b in range(3):
        fire(b, *bufs[b])

    def pipe(gg, _):
        for b in range(3):
            g = 3 * gg + b
            drain(*bufs[b])
            compute(g, bufs[b][0], bufs[b][1])
            fire(g + 3, *bufs[b])
        return 0

    n_loop = (NCHUNKS - 3) // 3
    lax.fori_loop(0, n_loop, pipe, 0)
    for g in range(3 * n_loop, NCHUNKS):
        b = g % 3
        drain(*bufs[b])
        compute(g, bufs[b][0], bufs[b][1])
        if g + 3 < NCHUNKS:
            fire(g + 3, *bufs[b])

    pltpu.sync_copy(obuf, out_hbm.at[pl.ds(base, RPW)])


_sc_dot = pl.kernel(
    _sc_body,
    out_type=jax.ShapeDtypeStruct((B,), jnp.float32),
    mesh=plsc.VectorSubcoreMesh(core_axis_name="c", subcore_axis_name="s"),
    scratch_types=[
        pltpu.VMEM((RPW,), jnp.int32),
        pltpu.VMEM((RPW,), jnp.int32),
        pltpu.VMEM((CHUNK, TDIM), jnp.float32),
        pltpu.VMEM((CHUNK, TDIM), jnp.float32),
        pltpu.VMEM((CHUNK, TDIM), jnp.float32),
        pltpu.VMEM((CHUNK, VWORDS), jnp.int32),
        pltpu.VMEM((CHUNK, VWORDS), jnp.int32),
        pltpu.VMEM((CHUNK, VWORDS), jnp.int32),
        pltpu.VMEM((RPW,), jnp.float32),
        pltpu.VMEM((LANES,), jnp.float32),
        pltpu.SemaphoreType.DMA,
        pltpu.SemaphoreType.DMA,
        pltpu.SemaphoreType.DMA,
        pltpu.SemaphoreType.DMA,
        pltpu.SemaphoreType.DMA,
        pltpu.SemaphoreType.DMA,
    ],
)


def kernel(P_weight, Q_weight, text_proj_W, classifier_W, classifier_b,
           query_ids, model_ids):
    v_packed = _build_v_table(P_weight, classifier_W, text_proj_W)
    b16 = jnp.broadcast_to(classifier_b.astype(jnp.float32), (LANES,))
    return _sc_dot(Q_weight, v_packed, b16, query_ids, model_ids)
